# baseline (speedup 1.0000x reference)
import functools

import jax
import jax.numpy as jnp
from jax import lax
from jax.experimental import pallas as pl
from jax.experimental.pallas import tpu as pltpu
from jax.experimental.pallas import tpu_sc as plsc

_B = 16384
_D = 64
_V = 1000000
_NSLAB = _V // 128 + 1
_TAIL_C = _NSLAB - 1
_TAIL_U = _TAIL_C * 128
_RING = 6
_PAIRS = 123


def _iota16():
    return lax.iota(jnp.int32, 16)


@jax.jit
def _gather_scan(idx, table_t, tail):
    mesh = plsc.VectorSubcoreMesh(core_axis_name="c", subcore_axis_name="s")

    @functools.partial(
        pl.kernel,
        mesh=mesh,
        out_type=jax.ShapeDtypeStruct((_B + 128, 128), jnp.float32),
        scratch_types=[
            pltpu.VMEM((_B,), jnp.int32),
            pltpu.VMEM((4096,), jnp.int32),
            pltpu.VMEM((256,), jnp.int32),
            pltpu.VMEM_SHARED((16, _RING, _D, 128), jnp.float32),
            pltpu.VMEM((_D, 128), jnp.float32),
            pltpu.VMEM((_D, 128), jnp.float32),
            pltpu.VMEM((128, 128), jnp.float32),
            pltpu.VMEM((128,), jnp.int32),
            pltpu.SemaphoreType.DMA,
            pltpu.SemaphoreType.DMA,
            pltpu.SemaphoreType.DMA,
        ],
        compiler_params=pltpu.CompilerParams(needs_layout_passes=False),
    )
    def k(idx_hbm, table_hbm, tail_hbm, out_hbm,
          idx_v, bucket_v, fill_v, sp_ring, t0, t1, stage_v, sl_v,
          sem_h, sem_x, sem_s):
        wid = lax.axis_index("s") * 2 + lax.axis_index("c")
        sid = lax.axis_index("s")
        c0 = 244 * wid + jnp.minimum(wid, 5)
        cnf = jnp.where(wid < 5, 245, jnp.where(wid == 31, 243, 244))
        cend = c0 + jnp.where(wid < 5, 245, 244)

        def fire_h(g):
            pltpu.async_copy(
                table_hbm.at[
                    :, pl.ds(pl.multiple_of((c0 + g) * 128, 128), 128)],
                sp_ring.at[sid, lax.rem(g, _RING)], sem_h)

        def wait_h():
            pltpu.make_async_copy(
                table_hbm.at[:, pl.ds(0, 128)],
                sp_ring.at[sid, 0], sem_h).wait()

        def fire_x(g, tbuf):
            pltpu.async_copy(sp_ring.at[sid, lax.rem(g, _RING)], tbuf, sem_x)

        def wait_x(tbuf):
            pltpu.make_async_copy(sp_ring.at[sid, 0], tbuf, sem_x).wait()

        for j in range(_RING):
            fire_h(jnp.int32(j))
        def init_sl(i, _):
            sl_v[pl.ds(i * 16, 16)] = jnp.full((16,), _B, jnp.int32)
            return 0
        lax.fori_loop(0, 8, init_sl, 0)

        def init_fill(i, _):
            fill_v[pl.ds(i * 16, 16)] = jnp.zeros((16,), jnp.int32)
            return 0
        lax.fori_loop(0, 16, init_fill, 0)

        pltpu.sync_copy(idx_hbm, idx_v)

        def bin_body(i, _):
            u = idx_v[pl.ds(i * 16, 16)]
            cc = lax.shift_right_logical(u, 7)
            mine = (cc >= c0) & (cc < cend)
            lb = cc - c0
            rank, last = plsc.scan_count(lb, mask=mine)
            base = plsc.load_gather(fill_v, [lb], mask=mine)
            slotb = base + rank - 1
            inb = mine & (slotb < 16)
            pack = (_iota16() + i * 16) | ((u & 127) << 14)
            plsc.store_scatter(bucket_v, [lb * 16 + slotb], pack, mask=inb)
            plsc.store_scatter(fill_v, [lb], base + rank, mask=mine & last)
            return 0

        lax.fori_loop(0, _B // 16, bin_body, 0)

        def ov_body(q, s):
            f16 = fill_v[pl.ds(q * 16, 16)]
            return s + jnp.sum(jnp.maximum(f16 - 16, 0))

        m_ov = lax.fori_loop(0, 16, ov_body, 0)

        def flush_reset(_sp):
            pltpu.async_copy(stage_v, out_hbm.at[sl_v], sem_s).wait()
            return 0

        def bucket_proc(lb, tbuf, sp):
            nh = plsc.load_gather(fill_v, [jnp.full((16,), 0, jnp.int32) + lb])
            sel = _iota16() < nh
            n = jnp.max(jnp.minimum(nh, 16))

            def do_group(sp):
                pp = bucket_v[pl.ds(lb * 16, 16)]
                bb = pp & 16383
                colv = lax.shift_right_logical(pp, 14)
                slot = sp + plsc.cumsum(sel.astype(jnp.int32)) - 1
                for f in range(_D):
                    fvec = jnp.full((16,), f, jnp.int32)
                    vals = plsc.load_gather(tbuf, [fvec, colv], mask=sel)
                    plsc.store_scatter(stage_v, [slot, fvec], vals, mask=sel)
                plsc.store_scatter(sl_v, [slot], bb, mask=sel)
                sp = sp + n
                return lax.cond(sp >= 112, flush_reset, lambda s: s, sp)

            return lax.cond(n > 0, do_group, lambda s: s, sp)

        wait_h()
        fire_x(jnp.int32(0), t0)

        def make_step(tbuf, tother):
            def step(g, sp):
                wait_x(tbuf)
                @pl.when(g + _RING < cnf)
                def _():
                    fire_h(g + _RING)

                @pl.when(g + 1 < cnf)
                def _():
                    wait_h()
                    fire_x(g + 1, tother)
                return bucket_proc(g, tbuf, sp)
            return step

        step0 = make_step(t0, t1)
        step1 = make_step(t1, t0)

        def pair_body(p, sp):
            sp = lax.cond(2 * p < cnf,
                          lambda s: step0(2 * p, s), lambda s: s, sp)
            sp = lax.cond(2 * p + 1 < cnf,
                          lambda s: step1(2 * p + 1, s), lambda s: s, sp)
            return sp

        sp = lax.fori_loop(0, _PAIRS, pair_body, 0)

        def do_tail(sp):
            pltpu.sync_copy(tail_hbm, t0)
            return bucket_proc(jnp.int32(243), t0, sp)

        sp = lax.cond(wid == 31, do_tail, lambda s: s, sp)

        def fallback(sp):
            def slab_body(jj, sp):
                c = c0 + jj

                @pl.when(c == _TAIL_C)
                def _():
                    pltpu.sync_copy(tail_hbm, t0)

                @pl.when(c != _TAIL_C)
                def _():
                    pltpu.sync_copy(
                        table_hbm.at[
                            :, pl.ds(pl.multiple_of(c * 128, 128), 128)],
                        t0)

                def qbody(q, sp):
                    uu = idx_v[pl.ds(q * 16, 16)]
                    sel = lax.shift_right_logical(uu, 7) == c
                    n = jnp.max(plsc.all_reduce_population_count(sel))

                    def do_group(sp):
                        bb = _iota16() + q * 16
                        colv = uu & 127
                        slot = sp + plsc.cumsum(sel.astype(jnp.int32)) - 1
                        for f in range(_D):
                            fvec = jnp.full((16,), f, jnp.int32)
                            vals = plsc.load_gather(
                                t0, [fvec, colv], mask=sel)
                            plsc.store_scatter(
                                stage_v, [slot, fvec], vals, mask=sel)
                        plsc.store_scatter(sl_v, [slot], bb, mask=sel)
                        sp = sp + n
                        return lax.cond(sp >= 112, flush_reset,
                                        lambda s: s, sp)

                    return lax.cond(n > 0, do_group, lambda s: s, sp)

                return lax.fori_loop(0, _B // 16, qbody, sp)

            return lax.fori_loop(0, cend - c0, slab_body, sp)

        sp = lax.cond(m_ov > 0, fallback, lambda s: s, sp)

        pltpu.async_copy(stage_v, out_hbm.at[sl_v], sem_s).wait()

    return k(idx, table_t, tail)


def kernel(user_indices, visual_tokens):
    B = user_indices.shape[0]
    V, T, D = visual_tokens.shape
    table_t = visual_tokens.reshape(V * T, D).T
    tail = jnp.pad(table_t[:, _TAIL_U:], ((0, 0), (0, 128 - (V - _TAIL_U))))
    idx = user_indices.astype(jnp.int32)
    out3 = _gather_scan(idx, table_t, tail)
    return out3[:B, :D].reshape(B, T, D)


# speedup vs baseline: 1.0001x; 1.0001x over previous
import functools

import jax
import jax.numpy as jnp
from jax import lax
from jax.experimental import pallas as pl
from jax.experimental.pallas import tpu as pltpu
from jax.experimental.pallas import tpu_sc as plsc

_B = 16384
_D = 64
_V = 1000000
_NSLAB = _V // 128 + 1
_TAIL_C = _NSLAB - 1
_TAIL_U = _TAIL_C * 128
_RING = 6
_PAIRS = 123


def _iota16():
    return lax.iota(jnp.int32, 16)


@jax.jit
def _gather_scan(idx, table_t, tail):
    mesh = plsc.VectorSubcoreMesh(core_axis_name="c", subcore_axis_name="s")

    @functools.partial(
        pl.kernel,
        mesh=mesh,
        out_type=jax.ShapeDtypeStruct((_B + 128, 128), jnp.float32),
        scratch_types=[
            pltpu.VMEM((_B,), jnp.int32),
            pltpu.VMEM((4096,), jnp.int32),
            pltpu.VMEM((256,), jnp.int32),
            pltpu.VMEM_SHARED((16, _RING, _D, 128), jnp.float32),
            pltpu.VMEM((_D, 128), jnp.float32),
            pltpu.VMEM((_D, 128), jnp.float32),
            pltpu.VMEM((128, 128), jnp.float32),
            pltpu.VMEM((128,), jnp.int32),
            pltpu.SemaphoreType.DMA((_RING,)),
            pltpu.SemaphoreType.DMA,
            pltpu.SemaphoreType.DMA,
        ],
        compiler_params=pltpu.CompilerParams(needs_layout_passes=False),
    )
    def k(idx_hbm, table_hbm, tail_hbm, out_hbm,
          idx_v, bucket_v, fill_v, sp_ring, t0, t1, stage_v, sl_v,
          sem_h, sem_x, sem_s):
        wid = lax.axis_index("s") * 2 + lax.axis_index("c")
        sid = lax.axis_index("s")
        c0 = 244 * wid + jnp.minimum(wid, 5)
        cnf = jnp.where(wid < 5, 245, jnp.where(wid == 31, 243, 244))
        cend = c0 + jnp.where(wid < 5, 245, 244)

        def fire_h(g):
            slot = lax.rem(g, _RING)
            pltpu.async_copy(
                table_hbm.at[
                    :, pl.ds(pl.multiple_of((c0 + g) * 128, 128), 128)],
                sp_ring.at[sid, slot], sem_h.at[slot])

        def wait_h(g):
            slot = lax.rem(g, _RING)
            pltpu.make_async_copy(
                table_hbm.at[:, pl.ds(0, 128)],
                sp_ring.at[sid, 0], sem_h.at[slot]).wait()

        def fire_x(g, tbuf):
            pltpu.async_copy(sp_ring.at[sid, lax.rem(g, _RING)], tbuf, sem_x)

        def wait_x(tbuf):
            pltpu.make_async_copy(sp_ring.at[sid, 0], tbuf, sem_x).wait()

        for j in range(_RING):
            fire_h(jnp.int32(j))
        def init_sl(i, _):
            sl_v[pl.ds(i * 16, 16)] = jnp.full((16,), _B, jnp.int32)
            return 0
        lax.fori_loop(0, 8, init_sl, 0)

        def init_fill(i, _):
            fill_v[pl.ds(i * 16, 16)] = jnp.zeros((16,), jnp.int32)
            return 0
        lax.fori_loop(0, 16, init_fill, 0)

        pltpu.sync_copy(idx_hbm, idx_v)

        def bin_body(i, _):
            u = idx_v[pl.ds(i * 16, 16)]
            cc = lax.shift_right_logical(u, 7)
            mine = (cc >= c0) & (cc < cend)
            lb = cc - c0
            rank, last = plsc.scan_count(lb, mask=mine)
            base = plsc.load_gather(fill_v, [lb], mask=mine)
            slotb = base + rank - 1
            inb = mine & (slotb < 16)
            pack = (_iota16() + i * 16) | ((u & 127) << 14)
            plsc.store_scatter(bucket_v, [lb * 16 + slotb], pack, mask=inb)
            plsc.store_scatter(fill_v, [lb], base + rank, mask=mine & last)
            return 0

        lax.fori_loop(0, _B // 16, bin_body, 0)

        def ov_body(q, s):
            f16 = fill_v[pl.ds(q * 16, 16)]
            return s + jnp.sum(jnp.maximum(f16 - 16, 0))

        m_ov = lax.fori_loop(0, 16, ov_body, 0)

        def flush_reset(_sp):
            pltpu.async_copy(stage_v, out_hbm.at[sl_v], sem_s).wait()
            return 0

        def bucket_proc(lb, tbuf, sp):
            nh = plsc.load_gather(fill_v, [jnp.full((16,), 0, jnp.int32) + lb])
            sel = _iota16() < nh
            n = jnp.max(jnp.minimum(nh, 16))

            def do_group(sp):
                pp = bucket_v[pl.ds(lb * 16, 16)]
                bb = pp & 16383
                colv = lax.shift_right_logical(pp, 14)
                slot = sp + plsc.cumsum(sel.astype(jnp.int32)) - 1
                for f in range(_D):
                    fvec = jnp.full((16,), f, jnp.int32)
                    vals = plsc.load_gather(tbuf, [fvec, colv], mask=sel)
                    plsc.store_scatter(stage_v, [slot, fvec], vals, mask=sel)
                plsc.store_scatter(sl_v, [slot], bb, mask=sel)
                sp = sp + n
                return lax.cond(sp >= 112, flush_reset, lambda s: s, sp)

            return lax.cond(n > 0, do_group, lambda s: s, sp)

        wait_h(jnp.int32(0))
        fire_x(jnp.int32(0), t0)

        def make_step(tbuf, tother):
            def step(g, sp):
                wait_x(tbuf)
                @pl.when(g + _RING < cnf)
                def _():
                    fire_h(g + _RING)

                @pl.when(g + 1 < cnf)
                def _():
                    wait_h(g + 1)
                    fire_x(g + 1, tother)
                return bucket_proc(g, tbuf, sp)
            return step

        step0 = make_step(t0, t1)
        step1 = make_step(t1, t0)

        def pair_body(p, sp):
            sp = lax.cond(2 * p < cnf,
                          lambda s: step0(2 * p, s), lambda s: s, sp)
            sp = lax.cond(2 * p + 1 < cnf,
                          lambda s: step1(2 * p + 1, s), lambda s: s, sp)
            return sp

        sp = lax.fori_loop(0, _PAIRS, pair_body, 0)

        def do_tail(sp):
            pltpu.sync_copy(tail_hbm, t0)
            return bucket_proc(jnp.int32(243), t0, sp)

        sp = lax.cond(wid == 31, do_tail, lambda s: s, sp)

        def fallback(sp):
            def slab_body(jj, sp):
                c = c0 + jj

                @pl.when(c == _TAIL_C)
                def _():
                    pltpu.sync_copy(tail_hbm, t0)

                @pl.when(c != _TAIL_C)
                def _():
                    pltpu.sync_copy(
                        table_hbm.at[
                            :, pl.ds(pl.multiple_of(c * 128, 128), 128)],
                        t0)

                def qbody(q, sp):
                    uu = idx_v[pl.ds(q * 16, 16)]
                    sel = lax.shift_right_logical(uu, 7) == c
                    n = jnp.max(plsc.all_reduce_population_count(sel))

                    def do_group(sp):
                        bb = _iota16() + q * 16
                        colv = uu & 127
                        slot = sp + plsc.cumsum(sel.astype(jnp.int32)) - 1
                        for f in range(_D):
                            fvec = jnp.full((16,), f, jnp.int32)
                            vals = plsc.load_gather(
                                t0, [fvec, colv], mask=sel)
                            plsc.store_scatter(
                                stage_v, [slot, fvec], vals, mask=sel)
                        plsc.store_scatter(sl_v, [slot], bb, mask=sel)
                        sp = sp + n
                        return lax.cond(sp >= 112, flush_reset,
                                        lambda s: s, sp)

                    return lax.cond(n > 0, do_group, lambda s: s, sp)

                return lax.fori_loop(0, _B // 16, qbody, sp)

            return lax.fori_loop(0, cend - c0, slab_body, sp)

        sp = lax.cond(m_ov > 0, fallback, lambda s: s, sp)

        pltpu.async_copy(stage_v, out_hbm.at[sl_v], sem_s).wait()

    return k(idx, table_t, tail)


def kernel(user_indices, visual_tokens):
    B = user_indices.shape[0]
    V, T, D = visual_tokens.shape
    table_t = visual_tokens.reshape(V * T, D).T
    tail = jnp.pad(table_t[:, _TAIL_U:], ((0, 0), (0, 128 - (V - _TAIL_U))))
    idx = user_indices.astype(jnp.int32)
    out3 = _gather_scan(idx, table_t, tail)
    return out3[:B, :D].reshape(B, T, D)
